# baseline (speedup 1.0000x reference)
import functools

import jax
import jax.numpy as jnp
from jax import lax
from jax.experimental import pallas as pl
from jax.experimental.pallas import tpu as pltpu
from jax.experimental.pallas import tpu_sc as plsc

L = 16


def _mesh_info():
    info = plsc.get_sparse_core_info()
    return info.num_cores, info.num_subcores


def _hist_body(nbins, rows_per_tile, nc, src_hbm, dst_hbm, out_hbm,
               sidx_v, didx_v, hs_v, hd_v):
    c = lax.axis_index("c")
    s = lax.axis_index("s")
    wid = s * nc + c

    def zero(i, _):
        z = jnp.zeros((L,), jnp.float32)
        hs_v[pl.ds(i * L, L)] = z
        hd_v[pl.ds(i * L, L)] = z
        return 0

    lax.fori_loop(0, nbins // L, zero, 0)

    pltpu.sync_copy(src_hbm.at[pl.ds(wid * rows_per_tile, rows_per_tile)],
                    sidx_v)
    pltpu.sync_copy(dst_hbm.at[pl.ds(wid * rows_per_tile, rows_per_tile)],
                    didx_v)

    ones = jnp.ones((L,), jnp.float32)

    def row(r, _):
        for g in range(128 // L):
            plsc.addupdate_scatter(hs_v, [sidx_v[r, pl.ds(g * L, L)]], ones)
            plsc.addupdate_scatter(hd_v, [didx_v[r, pl.ds(g * L, L)]], ones)
        return 0

    lax.fori_loop(0, rows_per_tile, row, 0)

    pltpu.sync_copy(hs_v, out_hbm.at[0, wid])
    pltpu.sync_copy(hd_v, out_hbm.at[1, wid])


def _agg_body(nbins, sup, rows_per_tile, nc, ns, half,
              h_hbm, src_hbm, dst_hbm, out_hbm,
              sidx_v, didx_v, buf0, buf1, acc_sh, sem0, sem1):
    c = lax.axis_index("c")
    s = lax.axis_index("s")
    bins_per_tile = nbins // ns
    slabs = bins_per_tile // 128
    hl = half // L
    groups = sup // L

    off = c * nbins
    brows = 8

    def load_block(bi):
        r0 = s * rows_per_tile + bi * brows
        pltpu.sync_copy(src_hbm.at[pl.ds(r0, brows)], sidx_v)
        pltpu.sync_copy(dst_hbm.at[pl.ds(r0, brows)], didx_v)

        def fix(r, _):
            for g in range(groups):
                sl = pl.ds(g * L, L)
                sidx_v[r, sl] = sidx_v[r, sl] + off
            return 0

        lax.fori_loop(0, brows, fix, 0)

    def zslab(k, _):
        buf0[k // hl, pl.ds((k % hl) * L, L)] = jnp.zeros((L,), jnp.float32)
        return 0

    lax.fori_loop(0, 128 * half // L, zslab, 0)

    plsc.subcore_barrier()

    cpb = brows

    def fire(ci, buf, sem):
        pltpu.async_copy(h_hbm.at[sidx_v.at[ci % cpb]], buf, sem)

    def drain(buf, sem):
        pltpu.make_async_copy(h_hbm.at[pl.ds(0, sup)], buf, sem).wait()

    def scat(ci, buf):
        return
        pltpu.sync_copy(buf, acc_sh.at[didx_v.at[ci % cpb]], add=True)

    n2 = rows_per_tile // 2
    load_block(0)
    fire(0, buf0, sem0)

    def body(it, _):
        i0 = 2 * it
        i1 = i0 + 1
        i2 = i0 + 2
        drain(buf0, sem0)
        fire(i1, buf1, sem1)
        scat(i0, buf0)
        drain(buf1, sem1)
        boundary = (i2 % cpb) == 0
        more = it < n2 - 1

        @pl.when(more & jnp.logical_not(boundary))
        def _():
            fire(i2, buf0, sem0)
            scat(i1, buf1)

        @pl.when(more & boundary)
        def _():
            scat(i1, buf1)
            load_block(i2 // cpb)
            fire(i2, buf0, sem0)

        @pl.when(jnp.logical_not(more))
        def _():
            scat(i1, buf1)

        return 0

    lax.fori_loop(0, n2, body, 0)
    plsc.subcore_barrier()

    for b in range(slabs):
        r0 = s * bins_per_tile + b * 128
        pltpu.sync_copy(acc_sh.at[pl.ds(r0, 128)],
                        out_hbm.at[c, pl.ds(r0, 128)])


def _prep_body(nc, hist_ref, feat_ref, h_ref, hsplit_ref, rin_ref):
    deg = jnp.sum(hist_ref[...], axis=-1, keepdims=True) + 1.0
    rs = lax.rsqrt(deg)
    h = feat_ref[...] * rs[0]
    h_ref[...] = h
    nbins, d = h.shape
    half = d // nc
    for i in range(nc):
        hsplit_ref[i * nbins:(i + 1) * nbins, :] = \
            h[:, i * half:(i + 1) * half]
    rin_ref[...] = rs[1]


def _final_body(nc, acc_ref, h_ref, rin_ref, out_ref):
    agg = jnp.concatenate([acc_ref[i] for i in range(nc)], axis=1)
    out_ref[...] = (agg + h_ref[...]) * rin_ref[...]


@jax.jit
def kernel(features, edge_index):
    n, d = features.shape
    e = edge_index.shape[1]
    nc, ns = _mesh_info()
    nw = nc * ns
    half = d // nc
    sup = 256

    nbins = ((n + 128 * ns) // (128 * ns)) * (128 * ns)
    nsup = -(-e // sup)
    rows_per_tile = -(-nsup // (ns * 8)) * 8
    nsup_p = rows_per_tile * ns
    epad = nsup_p * sup - e

    src = edge_index[0].astype(jnp.int32)
    dst = edge_index[1].astype(jnp.int32)
    sent = jnp.full((epad,), n, jnp.int32)
    src_f = jnp.concatenate([src, sent])
    dst_f = jnp.concatenate([dst, sent])
    src2d = src_f.reshape(nsup_p * sup // 128, 128)
    dst2d = dst_f.reshape(nsup_p * sup // 128, 128)
    srcs = src_f.reshape(nsup_p, sup)
    dsts = dst_f.reshape(nsup_p, sup)
    feat_p = jnp.pad(features, ((0, nbins - n), (0, 0)))
    erows_p = nsup_p * sup // 128

    mesh = plsc.VectorSubcoreMesh(core_axis_name="c", subcore_axis_name="s")
    sc_params = pltpu.CompilerParams(needs_layout_passes=False,
                                     use_tc_tiling_on_sc=False)

    hist = pl.kernel(
        functools.partial(_hist_body, nbins, erows_p // nw, nc),
        out_type=jax.ShapeDtypeStruct((2, nw, nbins), jnp.float32),
        mesh=mesh,
        scratch_types=[
            pltpu.VMEM((erows_p // nw, 128), jnp.int32),
            pltpu.VMEM((erows_p // nw, 128), jnp.int32),
            pltpu.VMEM((nbins,), jnp.float32),
            pltpu.VMEM((nbins,), jnp.float32),
        ],
        compiler_params=sc_params,
    )(src2d, dst2d)
    hist_t = jnp.transpose(hist, (0, 2, 1))

    h_p, hsplit, rin = pl.pallas_call(
        functools.partial(_prep_body, nc),
        out_shape=(
            jax.ShapeDtypeStruct((nbins, d), jnp.float32),
            jax.ShapeDtypeStruct((nc * nbins, half), jnp.float32),
            jax.ShapeDtypeStruct((nbins, 1), jnp.float32),
        ),
    )(hist_t, feat_p)

    acc = pl.kernel(
        functools.partial(_agg_body, nbins, sup, rows_per_tile, nc, ns,
                          half),
        out_type=jax.ShapeDtypeStruct((nc, nbins, half), jnp.float32),
        mesh=mesh,
        scratch_types=[
            pltpu.VMEM((8, sup), jnp.int32),
            pltpu.VMEM((8, sup), jnp.int32),
            pltpu.VMEM((sup, d), jnp.float32),
            pltpu.VMEM((sup, d), jnp.float32),
            pltpu.VMEM_SHARED((nbins, half), jnp.float32),
            pltpu.SemaphoreType.DMA,
            pltpu.SemaphoreType.DMA,
        ],
        compiler_params=sc_params,
    )(jnp.concatenate([h_p, h_p]), srcs, dsts)

    out = pl.pallas_call(
        functools.partial(_final_body, nc),
        out_shape=jax.ShapeDtypeStruct((nbins, d), jnp.float32),
    )(acc, h_p, rin)

    return out[:n]


# speedup vs baseline: 1.6176x; 1.6176x over previous
import functools

import jax
import jax.numpy as jnp
from jax import lax
from jax.experimental import pallas as pl
from jax.experimental.pallas import tpu as pltpu
from jax.experimental.pallas import tpu_sc as plsc

L = 16


def _mesh_info():
    info = plsc.get_sparse_core_info()
    return info.num_cores, info.num_subcores


def _hist_body(nbins, rows_per_tile, nc, ns, src_hbm, out_hbm,
               sidx_v, hs_v, tmp_v, red_v, stage_sh, sem):
    c = lax.axis_index("c")
    s = lax.axis_index("s")
    wid = s * nc + c
    bpt = nbins // ns

    def zero(i, _):
        hs_v[pl.ds(i * L, L)] = jnp.zeros((L,), jnp.float32)
        return 0

    lax.fori_loop(0, nbins // L, zero, 0)

    pltpu.sync_copy(src_hbm.at[pl.ds(wid * rows_per_tile, rows_per_tile)],
                    sidx_v)
    ones = jnp.ones((L,), jnp.float32)

    def row(r, _):
        for g in range(128 // L):
            plsc.addupdate_scatter(hs_v, [sidx_v[r, pl.ds(g * L, L)]], ones)
        return 0

    lax.fori_loop(0, rows_per_tile, row, 0)

    pltpu.sync_copy(hs_v, stage_sh.at[s])
    plsc.subcore_barrier()
    cp = pltpu.async_copy(stage_sh.at[:, pl.ds(s * bpt, bpt)], tmp_v, sem)
    cp.wait()

    def red(i, _):
        sl = pl.ds(i * L, L)
        v = tmp_v[0, sl]
        for t in range(1, ns):
            v = v + tmp_v[t, sl]
        red_v[sl] = v
        return 0

    lax.fori_loop(0, bpt // L, red, 0)
    pltpu.sync_copy(red_v, out_hbm.at[c, pl.ds(s * bpt, bpt)])


def _agg_body(nbins, sup, rows_per_tile, nc, ns, half,
              h_hbm, src_hbm, dst_hbm, acc_hbm, dh_hbm,
              sidx_v, didx_v, buf0, buf1, hd_v, acc_sh, sem0, sem1):
    c = lax.axis_index("c")
    s = lax.axis_index("s")
    bins_per_tile = nbins // ns
    slabs = bins_per_tile // 128
    hl = half // L
    groups = sup // L

    off = c * nbins
    brows = 8
    ones = jnp.ones((L,), jnp.float32)

    def zero(i, _):
        hd_v[pl.ds(i * L, L)] = jnp.zeros((L,), jnp.float32)
        return 0

    lax.fori_loop(0, nbins // L, zero, 0)

    def load_block(bi):
        r0 = s * rows_per_tile + bi * brows
        pltpu.sync_copy(src_hbm.at[pl.ds(r0, brows)], sidx_v)
        pltpu.sync_copy(dst_hbm.at[pl.ds(r0, brows)], didx_v)

        def fix(r, _):
            for g in range(groups):
                sl = pl.ds(g * L, L)
                sidx_v[r, sl] = sidx_v[r, sl] + off
                plsc.addupdate_scatter(hd_v, [didx_v[r, sl]], ones)
            return 0

        lax.fori_loop(0, brows, fix, 0)

    def zslab(k, _):
        buf0[k // hl, pl.ds((k % hl) * L, L)] = jnp.zeros((L,), jnp.float32)
        return 0

    lax.fori_loop(0, 128 * half // L, zslab, 0)

    for b in range(slabs):
        pltpu.sync_copy(buf0.at[pl.ds(0, 128)],
                        acc_sh.at[pl.ds(s * bins_per_tile + b * 128, 128)])
    plsc.subcore_barrier()

    cpb = brows

    def fire(ci, buf, sem):
        pltpu.async_copy(h_hbm.at[sidx_v.at[ci % cpb]], buf, sem)

    def drain(buf, sem):
        pltpu.make_async_copy(h_hbm.at[pl.ds(0, sup)], buf, sem).wait()

    def scat(ci, buf):
        pltpu.sync_copy(buf, acc_sh.at[didx_v.at[ci % cpb]], add=True)

    n2 = rows_per_tile // 2
    load_block(0)
    fire(0, buf0, sem0)

    def body(it, _):
        i0 = 2 * it
        i1 = i0 + 1
        i2 = i0 + 2
        drain(buf0, sem0)
        fire(i1, buf1, sem1)
        scat(i0, buf0)
        drain(buf1, sem1)
        boundary = (i2 % cpb) == 0
        more = it < n2 - 1

        @pl.when(more & jnp.logical_not(boundary))
        def _():
            fire(i2, buf0, sem0)
            scat(i1, buf1)

        @pl.when(more & boundary)
        def _():
            scat(i1, buf1)
            load_block(i2 // cpb)
            fire(i2, buf0, sem0)

        @pl.when(jnp.logical_not(more))
        def _():
            scat(i1, buf1)

        return 0

    lax.fori_loop(0, n2, body, 0)

    pltpu.sync_copy(hd_v, dh_hbm.at[s])
    plsc.subcore_barrier()

    for b in range(slabs):
        r0 = s * bins_per_tile + b * 128
        pltpu.sync_copy(acc_sh.at[pl.ds(r0, 128)],
                        acc_hbm.at[c, pl.ds(r0, 128)])


def _prep_body(n, nbins, nc, hist_ref, feat_ref, hsplit_ref):
    deg = hist_ref[:, 0:1] + hist_ref[:, 1:2] + 1.0
    rs = lax.rsqrt(deg)
    h = feat_ref[...] * rs[:n]
    d = h.shape[1]
    half = d // nc
    z = jnp.zeros((nbins - n, half), jnp.float32)
    for i in range(nc):
        hsplit_ref[i * nbins:i * nbins + n, :] = h[:, i * half:(i + 1) * half]
        hsplit_ref[i * nbins + n:(i + 1) * nbins, :] = z


def _final_body(n, nbins, nc, acc_ref, hsplit_ref, dh_ref, out_ref):
    deg_in = jnp.sum(dh_ref[:n], axis=1, keepdims=True) + 1.0
    rin = lax.rsqrt(deg_in)
    parts = [acc_ref[i, :n] + hsplit_ref[i * nbins:i * nbins + n]
             for i in range(nc)]
    out_ref[...] = jnp.concatenate(parts, axis=1) * rin


@jax.jit
def kernel(features, edge_index):
    n, d = features.shape
    e = edge_index.shape[1]
    nc, ns = _mesh_info()
    nw = nc * ns
    half = d // nc
    sup = 512

    nbins = ((n + 128 * ns) // (128 * ns)) * (128 * ns)
    nsup = -(-e // sup)
    rows_per_tile = -(-nsup // (ns * 8)) * 8
    nsup_p = rows_per_tile * ns
    epad = nsup_p * sup - e

    src = edge_index[0].astype(jnp.int32)
    dst = edge_index[1].astype(jnp.int32)
    sent = jnp.full((epad,), n, jnp.int32)
    src_f = jnp.concatenate([src, sent])
    dst_f = jnp.concatenate([dst, sent])
    erows_p = nsup_p * sup // 128
    src2d = src_f.reshape(erows_p, 128)
    srcs = src_f.reshape(nsup_p, sup)
    dsts = dst_f.reshape(nsup_p, sup)

    mesh = plsc.VectorSubcoreMesh(core_axis_name="c", subcore_axis_name="s")
    sc_params = pltpu.CompilerParams(needs_layout_passes=False,
                                     use_tc_tiling_on_sc=False)

    hout = pl.kernel(
        functools.partial(_hist_body, nbins, erows_p // nw, nc, ns),
        out_type=jax.ShapeDtypeStruct((nc, nbins), jnp.float32),
        mesh=mesh,
        scratch_types=[
            pltpu.VMEM((erows_p // nw, 128), jnp.int32),
            pltpu.VMEM((nbins,), jnp.float32),
            pltpu.VMEM((ns, nbins // ns), jnp.float32),
            pltpu.VMEM((nbins // ns,), jnp.float32),
            pltpu.VMEM_SHARED((ns, nbins), jnp.float32),
            pltpu.SemaphoreType.DMA,
        ],
        compiler_params=sc_params,
    )(src2d)
    hist_t = jnp.transpose(hout)

    hsplit = pl.pallas_call(
        functools.partial(_prep_body, n, nbins, nc),
        out_shape=jax.ShapeDtypeStruct((nc * nbins, half), jnp.float32),
    )(hist_t, features)

    acc, dh = pl.kernel(
        functools.partial(_agg_body, nbins, sup, rows_per_tile, nc, ns,
                          half),
        out_type=(
            jax.ShapeDtypeStruct((nc, nbins, half), jnp.float32),
            jax.ShapeDtypeStruct((ns, nbins), jnp.float32),
        ),
        mesh=mesh,
        scratch_types=[
            pltpu.VMEM((8, sup), jnp.int32),
            pltpu.VMEM((8, sup), jnp.int32),
            pltpu.VMEM((sup, half), jnp.float32),
            pltpu.VMEM((sup, half), jnp.float32),
            pltpu.VMEM((nbins,), jnp.float32),
            pltpu.VMEM_SHARED((nbins, half), jnp.float32),
            pltpu.SemaphoreType.DMA,
            pltpu.SemaphoreType.DMA,
        ],
        compiler_params=sc_params,
    )(hsplit, srcs, dsts)
    dh0 = jnp.transpose(dh)

    out = pl.pallas_call(
        functools.partial(_final_body, n, nbins, nc),
        out_shape=jax.ShapeDtypeStruct((n, d), jnp.float32),
    )(acc, hsplit, dh0)

    return out


# speedup vs baseline: 1.7521x; 1.0831x over previous
import functools

import jax
import jax.numpy as jnp
from jax import lax
from jax.experimental import pallas as pl
from jax.experimental.pallas import tpu as pltpu
from jax.experimental.pallas import tpu_sc as plsc

L = 16


def _mesh_info():
    info = plsc.get_sparse_core_info()
    return info.num_cores, info.num_subcores


def _hist_body(nbins, rows_per_tile, nc, ns, src_hbm, dst_hbm, out_hbm,
               sidx_v, didx_v, hs_v, hd_v, tmp_v, red_v, stage_sh, sem):
    c = lax.axis_index("c")
    s = lax.axis_index("s")
    wid = s * nc + c
    bpt = nbins // ns

    def zero(i, _):
        z = jnp.zeros((L,), jnp.float32)
        hs_v[pl.ds(i * L, L)] = z
        hd_v[pl.ds(i * L, L)] = z
        return 0

    lax.fori_loop(0, nbins // L, zero, 0)

    pltpu.sync_copy(src_hbm.at[pl.ds(wid * rows_per_tile, rows_per_tile)],
                    sidx_v)
    pltpu.sync_copy(dst_hbm.at[pl.ds(wid * rows_per_tile, rows_per_tile)],
                    didx_v)
    ones = jnp.ones((L,), jnp.float32)

    def row(r, _):
        for g in range(128 // L):
            sl = pl.ds(g * L, L)
            plsc.addupdate_scatter(hs_v, [sidx_v[r, sl]], ones)
            plsc.addupdate_scatter(hd_v, [didx_v[r, sl]], ones)
        return 0

    lax.fori_loop(0, rows_per_tile, row, 0)

    pltpu.sync_copy(hs_v, stage_sh.at[s, 0])
    pltpu.sync_copy(hd_v, stage_sh.at[s, 1])
    plsc.subcore_barrier()
    for k in range(2):
        cp = pltpu.async_copy(stage_sh.at[:, k, pl.ds(s * bpt, bpt)],
                              tmp_v, sem)
        cp.wait()

        def red(i, _):
            sl = pl.ds(i * L, L)
            v = tmp_v[0, sl]
            for t in range(1, ns):
                v = v + tmp_v[t, sl]
            red_v[sl] = v
            return 0

        lax.fori_loop(0, bpt // L, red, 0)
        pltpu.sync_copy(red_v, out_hbm.at[c, k, pl.ds(s * bpt, bpt)])


def _agg_body(nbins, sup, rows_per_tile, nc, ns, half,
              h_hbm, src_hbm, dst_hbm, acc_hbm,
              sidx_v, didx_v, buf0, buf1, acc_sh, sem0, sem1):
    c = lax.axis_index("c")
    s = lax.axis_index("s")
    bins_per_tile = nbins // ns
    slabs = bins_per_tile // 128
    hl = half // L
    groups = sup // L

    off = c * nbins
    brows = 8

    def load_block(bi):
        r0 = s * rows_per_tile + bi * brows
        pltpu.sync_copy(src_hbm.at[pl.ds(r0, brows)], sidx_v)
        pltpu.sync_copy(dst_hbm.at[pl.ds(r0, brows)], didx_v)

        def fix(r, _):
            for g in range(groups):
                sl = pl.ds(g * L, L)
                sidx_v[r, sl] = sidx_v[r, sl] + off
            return 0

        lax.fori_loop(0, brows, fix, 0)

    def zslab(k, _):
        buf0[k // hl, pl.ds((k % hl) * L, L)] = jnp.zeros((L,), jnp.float32)
        return 0

    lax.fori_loop(0, 128 * half // L, zslab, 0)

    for b in range(slabs):
        pltpu.sync_copy(buf0.at[pl.ds(0, 128)],
                        acc_sh.at[pl.ds(s * bins_per_tile + b * 128, 128)])
    plsc.subcore_barrier()

    cpb = brows

    def fire(ci, buf, sem):
        pltpu.async_copy(h_hbm.at[sidx_v.at[ci % cpb]], buf, sem)

    def drain(buf, sem):
        pltpu.make_async_copy(h_hbm.at[pl.ds(0, sup)], buf, sem).wait()

    def scat(ci, buf):
        pltpu.sync_copy(buf, acc_sh.at[didx_v.at[ci % cpb]], add=True)

    n2 = rows_per_tile // 2
    load_block(0)
    fire(0, buf0, sem0)

    def body(it, _):
        i0 = 2 * it
        i1 = i0 + 1
        i2 = i0 + 2
        drain(buf0, sem0)
        fire(i1, buf1, sem1)
        scat(i0, buf0)
        drain(buf1, sem1)
        boundary = (i2 % cpb) == 0
        more = it < n2 - 1

        @pl.when(more & jnp.logical_not(boundary))
        def _():
            fire(i2, buf0, sem0)
            scat(i1, buf1)

        @pl.when(more & boundary)
        def _():
            scat(i1, buf1)
            load_block(i2 // cpb)
            fire(i2, buf0, sem0)

        @pl.when(jnp.logical_not(more))
        def _():
            scat(i1, buf1)

        return 0

    lax.fori_loop(0, n2, body, 0)

    plsc.subcore_barrier()

    for b in range(slabs):
        r0 = s * bins_per_tile + b * 128
        pltpu.sync_copy(acc_sh.at[pl.ds(r0, 128)],
                        acc_hbm.at[c, pl.ds(r0, 128)])


def _prep_body(n, nbins, nc, hist_ref, feat_ref, hsplit_ref):
    deg = hist_ref[:, 0:1] + hist_ref[:, 1:2] + 1.0
    rs = lax.rsqrt(deg)
    h = feat_ref[...] * rs[:n]
    d = h.shape[1]
    half = d // nc
    z = jnp.zeros((nbins - n, half), jnp.float32)
    for i in range(nc):
        hsplit_ref[i * nbins:i * nbins + n, :] = h[:, i * half:(i + 1) * half]
        hsplit_ref[i * nbins + n:(i + 1) * nbins, :] = z


def _final_body(n, nbins, nc, acc_ref, hsplit_ref, dh_ref, out_ref):
    deg_in = dh_ref[:n, 0:1] + dh_ref[:n, 1:2] + 1.0
    rin = lax.rsqrt(deg_in)
    parts = [acc_ref[i, :n] + hsplit_ref[i * nbins:i * nbins + n]
             for i in range(nc)]
    out_ref[...] = jnp.concatenate(parts, axis=1) * rin


@jax.jit
def kernel(features, edge_index):
    n, d = features.shape
    e = edge_index.shape[1]
    nc, ns = _mesh_info()
    nw = nc * ns
    half = d // nc
    sup = 512

    nbins = ((n + 128 * ns) // (128 * ns)) * (128 * ns)
    nsup = -(-e // sup)
    rows_per_tile = -(-nsup // (ns * 8)) * 8
    nsup_p = rows_per_tile * ns
    epad = nsup_p * sup - e

    src = edge_index[0].astype(jnp.int32)
    dst = edge_index[1].astype(jnp.int32)
    sent = jnp.full((epad,), n, jnp.int32)
    src_f = jnp.concatenate([src, sent])
    dst_f = jnp.concatenate([dst, sent])
    erows_p = nsup_p * sup // 128
    src2d = src_f.reshape(erows_p, 128)
    dst2d = dst_f.reshape(erows_p, 128)
    srcs = src_f.reshape(nsup_p, sup)
    dsts = dst_f.reshape(nsup_p, sup)

    mesh = plsc.VectorSubcoreMesh(core_axis_name="c", subcore_axis_name="s")
    sc_params = pltpu.CompilerParams(needs_layout_passes=False,
                                     use_tc_tiling_on_sc=False)

    hout = pl.kernel(
        functools.partial(_hist_body, nbins, erows_p // nw, nc, ns),
        out_type=jax.ShapeDtypeStruct((nc, 2, nbins), jnp.float32),
        mesh=mesh,
        scratch_types=[
            pltpu.VMEM((erows_p // nw, 128), jnp.int32),
            pltpu.VMEM((erows_p // nw, 128), jnp.int32),
            pltpu.VMEM((nbins,), jnp.float32),
            pltpu.VMEM((nbins,), jnp.float32),
            pltpu.VMEM((ns, nbins // ns), jnp.float32),
            pltpu.VMEM((nbins // ns,), jnp.float32),
            pltpu.VMEM_SHARED((ns, 2, nbins), jnp.float32),
            pltpu.SemaphoreType.DMA,
        ],
        compiler_params=sc_params,
    )(src2d, dst2d)
    hist_t = jnp.transpose(hout[:, 0])
    dh_t = jnp.transpose(hout[:, 1])

    hsplit = pl.pallas_call(
        functools.partial(_prep_body, n, nbins, nc),
        out_shape=jax.ShapeDtypeStruct((nc * nbins, half), jnp.float32),
    )(hist_t, features)

    acc = pl.kernel(
        functools.partial(_agg_body, nbins, sup, rows_per_tile, nc, ns,
                          half),
        out_type=jax.ShapeDtypeStruct((nc, nbins, half), jnp.float32),
        mesh=mesh,
        scratch_types=[
            pltpu.VMEM((8, sup), jnp.int32),
            pltpu.VMEM((8, sup), jnp.int32),
            pltpu.VMEM((sup, half), jnp.float32),
            pltpu.VMEM((sup, half), jnp.float32),
            pltpu.VMEM_SHARED((nbins, half), jnp.float32),
            pltpu.SemaphoreType.DMA,
            pltpu.SemaphoreType.DMA,
        ],
        compiler_params=sc_params,
    )(hsplit, srcs, dsts)

    out = pl.pallas_call(
        functools.partial(_final_body, n, nbins, nc),
        out_shape=jax.ShapeDtypeStruct((n, d), jnp.float32),
    )(acc, hsplit, dh_t)

    return out


# speedup vs baseline: 1.8052x; 1.0303x over previous
import functools

import jax
import jax.numpy as jnp
from jax import lax
from jax.experimental import pallas as pl
from jax.experimental.pallas import tpu as pltpu
from jax.experimental.pallas import tpu_sc as plsc

L = 16


def _mesh_info():
    info = plsc.get_sparse_core_info()
    return info.num_cores, info.num_subcores


def _hist_body(nbins, rows_per_tile, nc, ns, src_hbm, dst_hbm, out_hbm,
               sidx_v, didx_v, hs_v, hd_v, tmp_v, red_v, stage_sh, sem):
    c = lax.axis_index("c")
    s = lax.axis_index("s")
    wid = s * nc + c
    bpt = nbins // ns

    def zero(i, _):
        z = jnp.zeros((L,), jnp.float32)
        hs_v[pl.ds(i * L, L)] = z
        hd_v[pl.ds(i * L, L)] = z
        return 0

    lax.fori_loop(0, nbins // L, zero, 0)

    pltpu.sync_copy(src_hbm.at[pl.ds(wid * rows_per_tile, rows_per_tile)],
                    sidx_v)
    pltpu.sync_copy(dst_hbm.at[pl.ds(wid * rows_per_tile, rows_per_tile)],
                    didx_v)
    ones = jnp.ones((L,), jnp.float32)

    def row(r, _):
        for g in range(128 // L):
            sl = pl.ds(g * L, L)
            plsc.addupdate_scatter(hs_v, [sidx_v[r, sl]], ones)
            plsc.addupdate_scatter(hd_v, [didx_v[r, sl]], ones)
        return 0

    lax.fori_loop(0, rows_per_tile, row, 0)

    pltpu.sync_copy(hs_v, stage_sh.at[s, 0])
    pltpu.sync_copy(hd_v, stage_sh.at[s, 1])
    plsc.subcore_barrier()
    for k in range(2):
        cp = pltpu.async_copy(stage_sh.at[:, k, pl.ds(s * bpt, bpt)],
                              tmp_v, sem)
        cp.wait()

        def red(i, _):
            sl = pl.ds(i * L, L)
            v = tmp_v[0, sl]
            for t in range(1, ns):
                v = v + tmp_v[t, sl]
            red_v[sl] = v
            return 0

        lax.fori_loop(0, bpt // L, red, 0)
        pltpu.sync_copy(red_v, out_hbm.at[c, k, pl.ds(s * bpt, bpt)])


def _agg_body(nbins, sup, rows_per_tile, nc, ns, half,
              h_hbm, src_hbm, dst_hbm, acc_hbm,
              si0, si1, di0, di1, b0, b1, b2, b3,
              acc_sh, s0, s1, s2, s3):
    c = lax.axis_index("c")
    s = lax.axis_index("s")
    bins_per_tile = nbins // ns
    slabs = bins_per_tile // 128
    hl = half // L
    groups = sup // L

    off = c * nbins
    brows = 4
    nchunks = rows_per_tile
    ngroups = nchunks // brows
    bufs = (b0, b1, b2, b3)
    sems = (s0, s1, s2, s3)

    def load_block(bi, si, di):
        r0 = s * rows_per_tile + bi * brows
        pltpu.sync_copy(src_hbm.at[pl.ds(r0, brows)], si)
        pltpu.sync_copy(dst_hbm.at[pl.ds(r0, brows)], di)

        def fix(r, _):
            for g in range(groups):
                sl = pl.ds(g * L, L)
                si[r, sl] = si[r, sl] + off
            return 0

        lax.fori_loop(0, brows, fix, 0)

    def zslab(k, _):
        b0[k // hl, pl.ds((k % hl) * L, L)] = jnp.zeros((L,), jnp.float32)
        return 0

    lax.fori_loop(0, 128 * half // L, zslab, 0)

    for b in range(slabs):
        pltpu.sync_copy(b0.at[pl.ds(0, 128)],
                        acc_sh.at[pl.ds(s * bins_per_tile + b * 128, 128)])
    plsc.subcore_barrier()

    def fire(ci, j, si):
        pltpu.async_copy(h_hbm.at[si.at[ci % brows]], bufs[j], sems[j])

    def drain(j):
        pltpu.make_async_copy(h_hbm.at[pl.ds(0, sup)], bufs[j],
                              sems[j]).wait()

    def scat(ci, j, di):
        pltpu.sync_copy(bufs[j], acc_sh.at[di.at[ci % brows]], add=True)

    load_block(0, si0, di0)
    fire(0, 0, si0)
    fire(1, 1, si0)
    fire(2, 2, si0)

    def body(it2, _):
        for p in range(2):
            it = 2 * it2 + p
            base = brows * it
            sip, dip = (si0, di0) if p == 0 else (si1, di1)
            siq, diq = (si1, di1) if p == 0 else (si0, di0)

            drain(0)
            scat(base, 0, dip)
            fire(base + 3, 3, sip)

            @pl.when(it + 1 < ngroups)
            def _():
                load_block(it + 1, siq, diq)

            for j in range(1, brows):
                drain(j)
                scat(base + j, j, dip)

                @pl.when(base + 3 + j < nchunks)
                def _(j=j):
                    fire(base + 3 + j, j - 1, siq)
        return 0

    lax.fori_loop(0, ngroups // 2, body, 0)

    plsc.subcore_barrier()

    for b in range(slabs):
        r0 = s * bins_per_tile + b * 128
        pltpu.sync_copy(acc_sh.at[pl.ds(r0, 128)],
                        acc_hbm.at[c, pl.ds(r0, 128)])


def _prep_body(n, nbins, nc, hist_ref, feat_ref, hsplit_ref):
    deg = hist_ref[:, 0:1] + hist_ref[:, 1:2] + 1.0
    rs = lax.rsqrt(deg)
    h = feat_ref[...] * rs[:n]
    d = h.shape[1]
    half = d // nc
    z = jnp.zeros((nbins - n, half), jnp.float32)
    for i in range(nc):
        hsplit_ref[i * nbins:i * nbins + n, :] = h[:, i * half:(i + 1) * half]
        hsplit_ref[i * nbins + n:(i + 1) * nbins, :] = z


def _final_body(n, nbins, nc, acc_ref, hsplit_ref, dh_ref, out_ref):
    deg_in = dh_ref[:n, 0:1] + dh_ref[:n, 1:2] + 1.0
    rin = lax.rsqrt(deg_in)
    parts = [acc_ref[i, :n] + hsplit_ref[i * nbins:i * nbins + n]
             for i in range(nc)]
    out_ref[...] = jnp.concatenate(parts, axis=1) * rin


@jax.jit
def kernel(features, edge_index):
    n, d = features.shape
    e = edge_index.shape[1]
    nc, ns = _mesh_info()
    nw = nc * ns
    half = d // nc
    sup = 256

    nbins = ((n + 128 * ns) // (128 * ns)) * (128 * ns)
    nsup = -(-e // sup)
    rows_per_tile = -(-nsup // (ns * 8)) * 8
    nsup_p = rows_per_tile * ns
    epad = nsup_p * sup - e

    src = edge_index[0].astype(jnp.int32)
    dst = edge_index[1].astype(jnp.int32)
    sent = jnp.full((epad,), n, jnp.int32)
    src_f = jnp.concatenate([src, sent])
    dst_f = jnp.concatenate([dst, sent])
    erows_p = nsup_p * sup // 128
    src2d = src_f.reshape(erows_p, 128)
    dst2d = dst_f.reshape(erows_p, 128)
    srcs = src_f.reshape(nsup_p, sup)
    dsts = dst_f.reshape(nsup_p, sup)

    mesh = plsc.VectorSubcoreMesh(core_axis_name="c", subcore_axis_name="s")
    sc_params = pltpu.CompilerParams(needs_layout_passes=False,
                                     use_tc_tiling_on_sc=False)

    hout = pl.kernel(
        functools.partial(_hist_body, nbins, erows_p // nw, nc, ns),
        out_type=jax.ShapeDtypeStruct((nc, 2, nbins), jnp.float32),
        mesh=mesh,
        scratch_types=[
            pltpu.VMEM((erows_p // nw, 128), jnp.int32),
            pltpu.VMEM((erows_p // nw, 128), jnp.int32),
            pltpu.VMEM((nbins,), jnp.float32),
            pltpu.VMEM((nbins,), jnp.float32),
            pltpu.VMEM((ns, nbins // ns), jnp.float32),
            pltpu.VMEM((nbins // ns,), jnp.float32),
            pltpu.VMEM_SHARED((ns, 2, nbins), jnp.float32),
            pltpu.SemaphoreType.DMA,
        ],
        compiler_params=sc_params,
    )(src2d, dst2d)
    hist_t = jnp.transpose(hout[:, 0])
    dh_t = jnp.transpose(hout[:, 1])

    hsplit = pl.pallas_call(
        functools.partial(_prep_body, n, nbins, nc),
        out_shape=jax.ShapeDtypeStruct((nc * nbins, half), jnp.float32),
    )(hist_t, features)

    acc = pl.kernel(
        functools.partial(_agg_body, nbins, sup, rows_per_tile, nc, ns,
                          half),
        out_type=jax.ShapeDtypeStruct((nc, nbins, half), jnp.float32),
        mesh=mesh,
        scratch_types=[
            pltpu.VMEM((4, sup), jnp.int32),
            pltpu.VMEM((4, sup), jnp.int32),
            pltpu.VMEM((4, sup), jnp.int32),
            pltpu.VMEM((4, sup), jnp.int32),
            pltpu.VMEM((sup, half), jnp.float32),
            pltpu.VMEM((sup, half), jnp.float32),
            pltpu.VMEM((sup, half), jnp.float32),
            pltpu.VMEM((sup, half), jnp.float32),
            pltpu.VMEM_SHARED((nbins, half), jnp.float32),
            pltpu.SemaphoreType.DMA,
            pltpu.SemaphoreType.DMA,
            pltpu.SemaphoreType.DMA,
            pltpu.SemaphoreType.DMA,
        ],
        compiler_params=sc_params,
    )(hsplit, srcs, dsts)

    out = pl.pallas_call(
        functools.partial(_final_body, n, nbins, nc),
        out_shape=jax.ShapeDtypeStruct((n, d), jnp.float32),
    )(acc, hsplit, dh_t)

    return out


# speedup vs baseline: 1.8153x; 1.0056x over previous
import functools

import jax
import jax.numpy as jnp
from jax import lax
from jax.experimental import pallas as pl
from jax.experimental.pallas import tpu as pltpu
from jax.experimental.pallas import tpu_sc as plsc

L = 16


def _mesh_info():
    info = plsc.get_sparse_core_info()
    return info.num_cores, info.num_subcores


def _hist_body(nbins, rows_per_tile, nc, ns, src_hbm, dst_hbm, out_hbm,
               sidx_v, didx_v, hs_v, hd_v, tmp_v, red_v, stage_sh, sem):
    c = lax.axis_index("c")
    s = lax.axis_index("s")
    wid = s * nc + c
    bpt = nbins // ns

    def zero(i, _):
        z = jnp.zeros((L,), jnp.float32)
        hs_v[pl.ds(i * L, L)] = z
        hd_v[pl.ds(i * L, L)] = z
        return 0

    lax.fori_loop(0, nbins // L, zero, 0)

    pltpu.sync_copy(src_hbm.at[pl.ds(wid * rows_per_tile, rows_per_tile)],
                    sidx_v)
    pltpu.sync_copy(dst_hbm.at[pl.ds(wid * rows_per_tile, rows_per_tile)],
                    didx_v)
    ones = jnp.ones((L,), jnp.float32)

    def row(r, _):
        for g in range(128 // L):
            sl = pl.ds(g * L, L)
            plsc.addupdate_scatter(hs_v, [sidx_v[r, sl]], ones)
            plsc.addupdate_scatter(hd_v, [didx_v[r, sl]], ones)
        return 0

    lax.fori_loop(0, rows_per_tile, row, 0)

    pltpu.sync_copy(hs_v, stage_sh.at[s, 0])
    pltpu.sync_copy(hd_v, stage_sh.at[s, 1])
    plsc.subcore_barrier()
    for k in range(2):
        cp = pltpu.async_copy(stage_sh.at[:, k, pl.ds(s * bpt, bpt)],
                              tmp_v, sem)
        cp.wait()

        def red(i, _):
            sl = pl.ds(i * L, L)
            v = tmp_v[0, sl]
            for t in range(1, ns):
                v = v + tmp_v[t, sl]
            red_v[sl] = v
            return 0

        lax.fori_loop(0, bpt // L, red, 0)
        pltpu.sync_copy(red_v, out_hbm.at[c, k, pl.ds(s * bpt, bpt)])


def _agg_body(nbins, sup, rows_per_tile, nc, ns, half,
              h_hbm, src_hbm, dst_hbm, acc_hbm,
              si0, si1, di0, di1, b0, b1, b2, b3, fbuf,
              acc_sh, s0, s1, s2, s3):
    c = lax.axis_index("c")
    s = lax.axis_index("s")
    bins_per_tile = nbins // ns
    slabs = bins_per_tile // 128
    hl = half // L
    groups = sup // L

    off = c * nbins
    brows = 4
    nchunks = rows_per_tile
    ngroups = nchunks // brows
    bufs = (b0, b1, b2, b3)
    sems = (s0, s1, s2, s3)

    def load_block(bi, si, di):
        r0 = s * rows_per_tile + bi * brows
        pltpu.sync_copy(src_hbm.at[pl.ds(r0, brows)], si)
        pltpu.sync_copy(dst_hbm.at[pl.ds(r0, brows)], di)

        def fix(r, _):
            for g in range(groups):
                sl = pl.ds(g * L, L)
                si[r, sl] = si[r, sl] + off
            return 0

        lax.fori_loop(0, brows, fix, 0)

    def zslab(k, _):
        fbuf[k // hl, pl.ds((k % hl) * L, L)] = jnp.zeros((L,), jnp.float32)
        return 0

    lax.fori_loop(0, 128 * half // L, zslab, 0)

    for b in range(slabs):
        pltpu.sync_copy(fbuf.at[pl.ds(0, 128)],
                        acc_sh.at[pl.ds(s * bins_per_tile + b * 128, 128)])
    plsc.subcore_barrier()

    def fire(ci, j, si):
        pltpu.async_copy(h_hbm.at[si.at[ci % brows]], bufs[j], sems[j])

    def drain(j):
        pltpu.make_async_copy(h_hbm.at[pl.ds(0, sup)], bufs[j],
                              sems[j]).wait()

    mask = jnp.int32(-65536)

    def expand(j):
        buf = bufs[j]

        def ex(r, _):
            for g in range(half // 32):
                x = buf[r, pl.ds(g * L, L)]
                lo = plsc.bitcast(lax.shift_left(x, 16), jnp.float32)
                hi = plsc.bitcast(lax.bitwise_and(x, mask), jnp.float32)
                fbuf[r, pl.ds(g * 32, L)] = lo
                fbuf[r, pl.ds(g * 32 + L, L)] = hi
            return 0

        lax.fori_loop(0, sup, ex, 0)

    def scat(ci, di):
        pltpu.sync_copy(fbuf, acc_sh.at[di.at[ci % brows]], add=True)

    load_block(0, si0, di0)
    fire(0, 0, si0)
    fire(1, 1, si0)
    fire(2, 2, si0)

    def body(it2, _):
        for p in range(2):
            it = 2 * it2 + p
            base = brows * it
            sip, dip = (si0, di0) if p == 0 else (si1, di1)
            siq, diq = (si1, di1) if p == 0 else (si0, di0)

            drain(0)
            fire(base + 3, 3, sip)
            expand(0)
            scat(base, dip)

            @pl.when(it + 1 < ngroups)
            def _():
                load_block(it + 1, siq, diq)

            for j in range(1, brows):
                drain(j)

                @pl.when(base + 3 + j < nchunks)
                def _(j=j):
                    fire(base + 3 + j, j - 1, siq)

                expand(j)
                scat(base + j, dip)
        return 0

    lax.fori_loop(0, ngroups // 2, body, 0)

    plsc.subcore_barrier()

    for b in range(slabs):
        r0 = s * bins_per_tile + b * 128
        pltpu.sync_copy(acc_sh.at[pl.ds(r0, 128)],
                        acc_hbm.at[c, pl.ds(r0, 128)])


def _pack_bf16(part):
    n, half = part.shape
    mask = jnp.int32(-65536)
    bits = lax.bitcast_convert_type(part, jnp.int32)
    words = []
    for g in range(half // 32):
        va = bits[:, g * 32:g * 32 + 16]
        vb = bits[:, g * 32 + 16:g * 32 + 32]
        words.append(lax.shift_right_logical(va, 16) | (vb & mask))
    return jnp.concatenate(words, axis=1)


def _prep_body(nc, hist_ref, feat_ref, hsplit_ref):
    deg = hist_ref[:, 0:1] + hist_ref[:, 1:2] + 1.0
    rs = lax.rsqrt(deg)
    h = feat_ref[...] * rs
    half = h.shape[1] // nc
    for i in range(nc):
        hsplit_ref[i] = _pack_bf16(h[:, i * half:(i + 1) * half])


def _unpack_bf16(packed):
    n, hw = packed.shape
    mask = jnp.int32(-65536)
    parts = []
    for g in range(hw // 16):
        w = packed[:, g * 16:(g + 1) * 16]
        parts.append(lax.bitcast_convert_type(lax.shift_left(w, 16),
                                              jnp.float32))
        parts.append(lax.bitcast_convert_type(w & mask, jnp.float32))
    return jnp.concatenate(parts, axis=1)


def _final_body(nc, acc_ref, hsplit_ref, dh_ref, out_ref):
    deg_in = dh_ref[:, 0:1] + dh_ref[:, 1:2] + 1.0
    rin = lax.rsqrt(deg_in)
    parts = [acc_ref[i] + _unpack_bf16(hsplit_ref[i]) for i in range(nc)]
    out_ref[...] = jnp.concatenate(parts, axis=1) * rin


@jax.jit
def kernel(features, edge_index):
    n, d = features.shape
    e = edge_index.shape[1]
    nc, ns = _mesh_info()
    nw = nc * ns
    half = d // nc
    sup = 256

    nbins = ((n + 128 * ns) // (128 * ns)) * (128 * ns)
    nsup = -(-e // sup)
    rows_per_tile = -(-nsup // (ns * 8)) * 8
    nsup_p = rows_per_tile * ns
    epad = nsup_p * sup - e

    src = edge_index[0].astype(jnp.int32)
    dst = edge_index[1].astype(jnp.int32)
    sent = jnp.full((epad,), n, jnp.int32)
    src_f = jnp.concatenate([src, sent])
    dst_f = jnp.concatenate([dst, sent])
    erows_p = nsup_p * sup // 128
    src2d = src_f.reshape(erows_p, 128)
    dst2d = dst_f.reshape(erows_p, 128)
    srcs = src_f.reshape(nsup_p, sup)
    dsts = dst_f.reshape(nsup_p, sup)

    mesh = plsc.VectorSubcoreMesh(core_axis_name="c", subcore_axis_name="s")
    sc_params = pltpu.CompilerParams(needs_layout_passes=False,
                                     use_tc_tiling_on_sc=False)

    hout = pl.kernel(
        functools.partial(_hist_body, nbins, erows_p // nw, nc, ns),
        out_type=jax.ShapeDtypeStruct((nc, 2, nbins), jnp.float32),
        mesh=mesh,
        scratch_types=[
            pltpu.VMEM((erows_p // nw, 128), jnp.int32),
            pltpu.VMEM((erows_p // nw, 128), jnp.int32),
            pltpu.VMEM((nbins,), jnp.float32),
            pltpu.VMEM((nbins,), jnp.float32),
            pltpu.VMEM((ns, nbins // ns), jnp.float32),
            pltpu.VMEM((nbins // ns,), jnp.float32),
            pltpu.VMEM_SHARED((ns, 2, nbins), jnp.float32),
            pltpu.SemaphoreType.DMA,
        ],
        compiler_params=sc_params,
    )(src2d, dst2d)
    hist_t = jnp.transpose(hout[:, 0])
    dh_t = jnp.transpose(hout[:, 1])
    feat_p = jnp.pad(features, ((0, nbins - n), (0, 0)))
    hw = half // 2
    bs = 1024

    hsplit3 = pl.pallas_call(
        functools.partial(_prep_body, nc),
        grid=(nbins // bs,),
        in_specs=[
            pl.BlockSpec((bs, 2), lambda i: (i, 0)),
            pl.BlockSpec((bs, d), lambda i: (i, 0)),
        ],
        out_specs=pl.BlockSpec((nc, bs, hw), lambda i: (0, i, 0)),
        out_shape=jax.ShapeDtypeStruct((nc, nbins, hw), jnp.int32),
    )(hist_t, feat_p)
    hsplit = hsplit3.reshape(nc * nbins, hw)

    acc = pl.kernel(
        functools.partial(_agg_body, nbins, sup, rows_per_tile, nc, ns,
                          half),
        out_type=jax.ShapeDtypeStruct((nc, nbins, half), jnp.float32),
        mesh=mesh,
        scratch_types=[
            pltpu.VMEM((4, sup), jnp.int32),
            pltpu.VMEM((4, sup), jnp.int32),
            pltpu.VMEM((4, sup), jnp.int32),
            pltpu.VMEM((4, sup), jnp.int32),
            pltpu.VMEM((sup, half // 2), jnp.int32),
            pltpu.VMEM((sup, half // 2), jnp.int32),
            pltpu.VMEM((sup, half // 2), jnp.int32),
            pltpu.VMEM((sup, half // 2), jnp.int32),
            pltpu.VMEM((sup, half), jnp.float32),
            pltpu.VMEM_SHARED((nbins, half), jnp.float32),
            pltpu.SemaphoreType.DMA,
            pltpu.SemaphoreType.DMA,
            pltpu.SemaphoreType.DMA,
            pltpu.SemaphoreType.DMA,
        ],
        compiler_params=sc_params,
    )(hsplit, srcs, dsts)

    bs2 = 1000 if n % 1000 == 0 else n
    out = pl.pallas_call(
        functools.partial(_final_body, nc),
        grid=(n // bs2,),
        in_specs=[
            pl.BlockSpec((nc, bs2, half), lambda i: (0, i, 0)),
            pl.BlockSpec((nc, bs2, hw), lambda i: (0, i, 0)),
            pl.BlockSpec((bs2, 2), lambda i: (i, 0)),
        ],
        out_specs=pl.BlockSpec((bs2, d), lambda i: (i, 0)),
        out_shape=jax.ShapeDtypeStruct((n, d), jnp.float32),
    )(acc, hsplit3, dh_t)

    return out


# speedup vs baseline: 2.0094x; 1.1069x over previous
import functools

import jax
import jax.numpy as jnp
from jax import lax
from jax.experimental import pallas as pl
from jax.experimental.pallas import tpu as pltpu
from jax.experimental.pallas import tpu_sc as plsc

L = 16


def _mesh_info():
    info = plsc.get_sparse_core_info()
    return info.num_cores, info.num_subcores


def _hist_body(nbins, rows_per_tile, nc, ns, src_hbm, dst_hbm, out_hbm,
               sidx_v, didx_v, hs_v, hd_v, tmp_v, red_v, stage_sh, sem):
    c = lax.axis_index("c")
    s = lax.axis_index("s")
    wid = s * nc + c
    bpt = nbins // ns

    def zero(i, _):
        z = jnp.zeros((L,), jnp.float32)
        hs_v[pl.ds(i * L, L)] = z
        hd_v[pl.ds(i * L, L)] = z
        return 0

    lax.fori_loop(0, nbins // L, zero, 0)

    pltpu.sync_copy(src_hbm.at[pl.ds(wid * rows_per_tile, rows_per_tile)],
                    sidx_v)
    pltpu.sync_copy(dst_hbm.at[pl.ds(wid * rows_per_tile, rows_per_tile)],
                    didx_v)
    ones = jnp.ones((L,), jnp.float32)

    def row(r, _):
        for g in range(128 // L):
            sl = pl.ds(g * L, L)
            plsc.addupdate_scatter(hs_v, [sidx_v[r, sl]], ones)
            plsc.addupdate_scatter(hd_v, [didx_v[r, sl]], ones)
        return 0

    lax.fori_loop(0, rows_per_tile, row, 0)

    pltpu.sync_copy(hs_v, stage_sh.at[s, 0])
    pltpu.sync_copy(hd_v, stage_sh.at[s, 1])
    plsc.subcore_barrier()
    for k in range(2):
        cp = pltpu.async_copy(stage_sh.at[:, k, pl.ds(s * bpt, bpt)],
                              tmp_v, sem)
        cp.wait()

        def red(i, _):
            sl = pl.ds(i * L, L)
            v = tmp_v[0, sl]
            for t in range(1, ns):
                v = v + tmp_v[t, sl]
            red_v[sl] = v
            return 0

        lax.fori_loop(0, bpt // L, red, 0)
        pltpu.sync_copy(red_v, out_hbm.at[c, k, pl.ds(s * bpt, bpt)])


def _agg_body(nbins, sup, rows_per_tile, nc, ns, half,
              h_hbm, src_hbm, dst_hbm, acc_hbm,
              si0, si1, di0, di1, b0, b1, b2, b3, fbuf0, fbuf1,
              acc_sh, s0, s1, s2, s3, ss0, ss1):
    c = lax.axis_index("c")
    s = lax.axis_index("s")
    bins_per_tile = nbins // ns
    slabs = bins_per_tile // 128
    hl = half // L
    groups = sup // L

    off = c * nbins
    brows = 4
    nchunks = rows_per_tile
    ngroups = nchunks // brows
    bufs = (b0, b1, b2, b3)
    sems = (s0, s1, s2, s3)

    def load_block(bi, si, di):
        r0 = s * rows_per_tile + bi * brows
        pltpu.sync_copy(src_hbm.at[pl.ds(r0, brows)], si)
        pltpu.sync_copy(dst_hbm.at[pl.ds(r0, brows)], di)

        def fix(r, _):
            for g in range(groups):
                sl = pl.ds(g * L, L)
                si[r, sl] = si[r, sl] + off
            return 0

        lax.fori_loop(0, brows, fix, 0)

    fbufs = (fbuf0, fbuf1)
    ssems = (ss0, ss1)

    def zslab(k, _):
        fbuf0[k // hl, pl.ds((k % hl) * L, L)] = jnp.zeros((L,),
                                                           jnp.float32)
        return 0

    lax.fori_loop(0, 128 * half // L, zslab, 0)

    for b in range(slabs):
        pltpu.sync_copy(fbuf0.at[pl.ds(0, 128)],
                        acc_sh.at[pl.ds(s * bins_per_tile + b * 128, 128)])
    plsc.subcore_barrier()

    def fire(ci, j, si):
        pltpu.async_copy(h_hbm.at[si.at[ci % brows]], bufs[j], sems[j])

    def drain(j):
        pltpu.make_async_copy(h_hbm.at[pl.ds(0, sup)], bufs[j],
                              sems[j]).wait()

    mask = jnp.int32(-65536)

    def expand(j, p):
        buf = bufs[j]
        fb = fbufs[p]

        def ex(r, _):
            for g in range(half // 32):
                x = buf[r, pl.ds(g * L, L)]
                lo = plsc.bitcast(lax.shift_left(x, 16), jnp.float32)
                hi = plsc.bitcast(lax.bitwise_and(x, mask), jnp.float32)
                fb[r, pl.ds(g * 32, L)] = lo
                fb[r, pl.ds(g * 32 + L, L)] = hi
            return 0

        lax.fori_loop(0, sup, ex, 0)

    def scat_fire(ci, p, di):
        pltpu.async_copy(fbufs[p], acc_sh.at[di.at[ci % brows]], ssems[p],
                         add=True)

    def scat_drain(p):
        pltpu.make_async_copy(fbufs[p], acc_sh.at[pl.ds(0, sup)],
                              ssems[p]).wait()

    load_block(0, si0, di0)
    fire(0, 0, si0)
    fire(1, 1, si0)
    fire(2, 2, si0)

    def body(it2, _):
        for p in range(2):
            it = 2 * it2 + p
            base = brows * it
            sip, dip = (si0, di0) if p == 0 else (si1, di1)
            siq, diq = (si1, di1) if p == 0 else (si0, di0)

            drain(0)
            fire(base + 3, 3, sip)

            @pl.when(base >= 2)
            def _():
                scat_drain(0)

            expand(0, 0)
            scat_fire(base, 0, dip)

            @pl.when(it + 1 < ngroups)
            def _():
                load_block(it + 1, siq, diq)

            for j in range(1, brows):
                drain(j)

                @pl.when(base + 3 + j < nchunks)
                def _(j=j):
                    fire(base + 3 + j, j - 1, siq)

                @pl.when(base + j >= 2)
                def _(j=j):
                    scat_drain(j % 2)

                expand(j, j % 2)
                scat_fire(base + j, j % 2, dip)
        return 0

    lax.fori_loop(0, ngroups // 2, body, 0)
    scat_drain(0)
    scat_drain(1)

    plsc.subcore_barrier()

    for b in range(slabs):
        r0 = s * bins_per_tile + b * 128
        pltpu.sync_copy(acc_sh.at[pl.ds(r0, 128)],
                        acc_hbm.at[c, pl.ds(r0, 128)])


def _pack_bf16(part):
    n, half = part.shape
    mask = jnp.int32(-65536)
    bits = lax.bitcast_convert_type(part, jnp.int32)
    words = []
    for g in range(half // 32):
        va = bits[:, g * 32:g * 32 + 16]
        vb = bits[:, g * 32 + 16:g * 32 + 32]
        words.append(lax.shift_right_logical(va, 16) | (vb & mask))
    return jnp.concatenate(words, axis=1)


def _prep_body(nc, hist_ref, feat_ref, hsplit_ref):
    deg = hist_ref[:, 0:1] + hist_ref[:, 1:2] + 1.0
    rs = lax.rsqrt(deg)
    h = feat_ref[...] * rs
    half = h.shape[1] // nc
    for i in range(nc):
        hsplit_ref[i] = _pack_bf16(h[:, i * half:(i + 1) * half])


def _unpack_bf16(packed):
    n, hw = packed.shape
    mask = jnp.int32(-65536)
    parts = []
    for g in range(hw // 16):
        w = packed[:, g * 16:(g + 1) * 16]
        parts.append(lax.bitcast_convert_type(lax.shift_left(w, 16),
                                              jnp.float32))
        parts.append(lax.bitcast_convert_type(w & mask, jnp.float32))
    return jnp.concatenate(parts, axis=1)


def _final_body(nc, acc_ref, hsplit_ref, dh_ref, out_ref):
    deg_in = dh_ref[:, 0:1] + dh_ref[:, 1:2] + 1.0
    rin = lax.rsqrt(deg_in)
    parts = [acc_ref[i] + _unpack_bf16(hsplit_ref[i]) for i in range(nc)]
    out_ref[...] = jnp.concatenate(parts, axis=1) * rin


@jax.jit
def kernel(features, edge_index):
    n, d = features.shape
    e = edge_index.shape[1]
    nc, ns = _mesh_info()
    nw = nc * ns
    half = d // nc
    sup = 256

    nbins = ((n + 128 * ns) // (128 * ns)) * (128 * ns)
    nsup = -(-e // sup)
    rows_per_tile = -(-nsup // (ns * 8)) * 8
    nsup_p = rows_per_tile * ns
    epad = nsup_p * sup - e

    src = edge_index[0].astype(jnp.int32)
    dst = edge_index[1].astype(jnp.int32)
    sent = jnp.full((epad,), n, jnp.int32)
    src_f = jnp.concatenate([src, sent])
    dst_f = jnp.concatenate([dst, sent])
    erows_p = nsup_p * sup // 128
    src2d = src_f.reshape(erows_p, 128)
    dst2d = dst_f.reshape(erows_p, 128)
    srcs = src_f.reshape(nsup_p, sup)
    dsts = dst_f.reshape(nsup_p, sup)

    mesh = plsc.VectorSubcoreMesh(core_axis_name="c", subcore_axis_name="s")
    sc_params = pltpu.CompilerParams(needs_layout_passes=False,
                                     use_tc_tiling_on_sc=False)

    hout = pl.kernel(
        functools.partial(_hist_body, nbins, erows_p // nw, nc, ns),
        out_type=jax.ShapeDtypeStruct((nc, 2, nbins), jnp.float32),
        mesh=mesh,
        scratch_types=[
            pltpu.VMEM((erows_p // nw, 128), jnp.int32),
            pltpu.VMEM((erows_p // nw, 128), jnp.int32),
            pltpu.VMEM((nbins,), jnp.float32),
            pltpu.VMEM((nbins,), jnp.float32),
            pltpu.VMEM((ns, nbins // ns), jnp.float32),
            pltpu.VMEM((nbins // ns,), jnp.float32),
            pltpu.VMEM_SHARED((ns, 2, nbins), jnp.float32),
            pltpu.SemaphoreType.DMA,
        ],
        compiler_params=sc_params,
    )(src2d, dst2d)
    hist_t = jnp.transpose(hout[:, 0])
    dh_t = jnp.transpose(hout[:, 1])
    feat_p = jnp.pad(features, ((0, nbins - n), (0, 0)))
    hw = half // 2
    bs = 1024

    hsplit3 = pl.pallas_call(
        functools.partial(_prep_body, nc),
        grid=(nbins // bs,),
        in_specs=[
            pl.BlockSpec((bs, 2), lambda i: (i, 0)),
            pl.BlockSpec((bs, d), lambda i: (i, 0)),
        ],
        out_specs=pl.BlockSpec((nc, bs, hw), lambda i: (0, i, 0)),
        out_shape=jax.ShapeDtypeStruct((nc, nbins, hw), jnp.int32),
    )(hist_t, feat_p)
    hsplit = hsplit3.reshape(nc * nbins, hw)

    acc = pl.kernel(
        functools.partial(_agg_body, nbins, sup, rows_per_tile, nc, ns,
                          half),
        out_type=jax.ShapeDtypeStruct((nc, nbins, half), jnp.float32),
        mesh=mesh,
        scratch_types=[
            pltpu.VMEM((4, sup), jnp.int32),
            pltpu.VMEM((4, sup), jnp.int32),
            pltpu.VMEM((4, sup), jnp.int32),
            pltpu.VMEM((4, sup), jnp.int32),
            pltpu.VMEM((sup, half // 2), jnp.int32),
            pltpu.VMEM((sup, half // 2), jnp.int32),
            pltpu.VMEM((sup, half // 2), jnp.int32),
            pltpu.VMEM((sup, half // 2), jnp.int32),
            pltpu.VMEM((sup, half), jnp.float32),
            pltpu.VMEM((sup, half), jnp.float32),
            pltpu.VMEM_SHARED((nbins, half), jnp.float32),
            pltpu.SemaphoreType.DMA,
            pltpu.SemaphoreType.DMA,
            pltpu.SemaphoreType.DMA,
            pltpu.SemaphoreType.DMA,
            pltpu.SemaphoreType.DMA,
            pltpu.SemaphoreType.DMA,
        ],
        compiler_params=sc_params,
    )(hsplit, srcs, dsts)

    bs2 = 1000 if n % 1000 == 0 else n
    out = pl.pallas_call(
        functools.partial(_final_body, nc),
        grid=(n // bs2,),
        in_specs=[
            pl.BlockSpec((nc, bs2, half), lambda i: (0, i, 0)),
            pl.BlockSpec((nc, bs2, hw), lambda i: (0, i, 0)),
            pl.BlockSpec((bs2, 2), lambda i: (i, 0)),
        ],
        out_specs=pl.BlockSpec((bs2, d), lambda i: (i, 0)),
        out_shape=jax.ShapeDtypeStruct((n, d), jnp.float32),
    )(acc, hsplit3, dh_t)

    return out
